# baseline (speedup 1.0000x reference)
import functools

import jax
import jax.numpy as jnp
from jax import lax
from jax.experimental import pallas as pl
from jax.experimental.pallas import tpu as pltpu
from jax.experimental.pallas import tpu_sc as plsc

_NUM_CORES = 2
_NUM_SUBCORES = 16
_NW = _NUM_CORES * _NUM_SUBCORES
_IDS_PER_B = 64


def _fuse_kernel(t_ref, w_ref, b_ref, o_ref):
    o_ref[...] = (
        jnp.dot(t_ref[...], w_ref[...], preferred_element_type=jnp.float32)
        + b_ref[...]
    )


def _fuse_table(embed_table, proj_w, proj_b):
    V, E = embed_table.shape
    VO = proj_w.shape[1]
    D = 128
    VP = 1024
    t_pad = jnp.pad(embed_table, ((0, VP - V), (0, D - E)))
    w_pad = jnp.pad(proj_w, ((0, D - E), (0, 0)))
    return pl.pallas_call(
        _fuse_kernel,
        out_shape=jax.ShapeDtypeStruct((VP, VO), jnp.float32),
    )(t_pad, w_pad, proj_b.reshape(1, VO))


def _make_gather(B, L, V, VO):
    b_per_w = B // _NW
    ids_per_w = b_per_w * _IDS_PER_B
    elems_per_sub = V * VO // _NUM_SUBCORES
    mesh = plsc.VectorSubcoreMesh(core_axis_name="c", subcore_axis_name="s")

    @functools.partial(
        pl.kernel,
        out_type=jax.ShapeDtypeStruct((B, L, VO), jnp.float32),
        mesh=mesh,
        scratch_types=[
            pltpu.VMEM_SHARED((V, VO), jnp.float32),
            pltpu.VMEM((ids_per_w,), jnp.int32),
            pltpu.VMEM((L, VO), jnp.float32),
            pltpu.SemaphoreType.DMA,
        ],
        compiler_params=pltpu.CompilerParams(use_tc_tiling_on_sc=False),
    )
    def gather(p_hbm, ids_hbm, out_hbm, p_sp, idx_v, buf, sem):
        cid = lax.axis_index("c")
        sid = lax.axis_index("s")
        wid = sid * _NUM_CORES + cid
        r0 = sid * (V // _NUM_SUBCORES)
        pltpu.sync_copy(p_hbm.at[pl.ds(r0, V // _NUM_SUBCORES)],
                        p_sp.at[pl.ds(r0, V // _NUM_SUBCORES)])
        plsc.subcore_barrier()
        pltpu.sync_copy(ids_hbm.at[pl.ds(wid * ids_per_w, ids_per_w)], idx_v)
        b0 = wid * b_per_w

        def body(j, carry):
            pltpu.async_copy(
                p_sp.at[idx_v.at[pl.ds(j * _IDS_PER_B, L)]], buf, sem
            ).wait()
            pltpu.sync_copy(buf, out_hbm.at[b0 + j])
            return carry

        lax.fori_loop(0, b_per_w, body, 0)

    return gather


def kernel(input_ids, embed_table, proj_w, proj_b):
    B, L = input_ids.shape
    V = embed_table.shape[0]
    VO = proj_w.shape[1]

    P = _fuse_table(embed_table, proj_w, proj_b)
    ids_pad = jnp.pad(input_ids.astype(jnp.int32),
                      ((0, 0), (0, _IDS_PER_B - L)))
    return _make_gather(B, L, P.shape[0], VO)(P, ids_pad.reshape(B * _IDS_PER_B))
